# baseline (speedup 1.0000x reference)
import functools

import jax
import jax.numpy as jnp
from jax import lax
from jax.experimental import pallas as pl
from jax.experimental.pallas import tpu as pltpu
from jax.experimental.pallas import tpu_sc as plsc


@functools.lru_cache(maxsize=None)
def _make_gather(n_rows: int, n_tags: int, d: int, seq: int):
    info = plsc.get_sparse_core_info()
    nc, ns = info.num_cores, info.num_subcores
    nw = nc * ns
    per_w = n_rows // nw
    g = 128 // d
    sg = seq // g
    chunk = 128 * g
    sub = 128
    n_sub = chunk // sub
    assert per_w == sg * chunk and sg % 2 == 0

    mesh = plsc.VectorSubcoreMesh(core_axis_name="c", subcore_axis_name="s")

    @functools.partial(
        pl.kernel,
        mesh=mesh,
        out_type=jax.ShapeDtypeStruct((n_rows, d), jnp.float32),
        scratch_types=[
            pltpu.VMEM((per_w,), jnp.int32),
            pltpu.VMEM((per_w,), jnp.int32),
            pltpu.VMEM((chunk, d), jnp.float32),
            pltpu.VMEM((chunk, d), jnp.float32),
            pltpu.VMEM((chunk, d), jnp.float32),
            pltpu.VMEM((chunk, d), jnp.float32),
            pltpu.SemaphoreType.DMA,
            pltpu.SemaphoreType.DMA,
            pltpu.SemaphoreType.DMA,
            pltpu.SemaphoreType.DMA,
            pltpu.SemaphoreType.DMA,
            pltpu.SemaphoreType.DMA,
            pltpu.SemaphoreType.DMA,
            pltpu.SemaphoreType.DMA,
        ],
        compiler_params=pltpu.CompilerParams(
            use_tc_tiling_on_sc=False, needs_layout_passes=False),
    )
    def k(x_hbm, tab_hbm, out_hbm, xblk, idx_all, rows0, rows1, rows2, rows3,
          sg0, sg1, sg2, sg3, so0, so1, so2, so3):
        rows_v = (rows0, rows1, rows2, rows3)
        sem_gat = (sg0, sg1, sg2, sg3)
        sem_out = (so0, so1, so2, so3)

        wid = lax.axis_index("s") * nc + lax.axis_index("c")
        base_w = wid * per_w

        pltpu.sync_copy(x_hbm.at[pl.ds(base_w, per_w)], xblk)

        iota = lax.iota(jnp.int32, 16)
        base_off = (iota >> 2) * seq + (iota & (g - 1))

        def build_group(c):
            def bg(v, carry):
                off = base_off + v * (4 * seq) + c * g
                vals = plsc.load_gather(xblk, [off])
                idx_all[pl.ds(c * chunk + v * 16, 16)] = vals
                return carry

            lax.fori_loop(0, chunk // 16, bg, 0)

        def gather_copy(c, b, j):
            return pltpu.make_async_copy(
                tab_hbm.at[idx_all.at[pl.ds(c * chunk + j * sub, sub)]],
                rows_v[b].at[pl.ds(j * sub, sub)], sem_gat[b])

        def store_copy(c, b):
            return pltpu.make_async_copy(
                rows_v[b], out_hbm.at[pl.ds(base_w + c * chunk, chunk)],
                sem_out[b])

        def fire(c, b):
            for j in range(n_sub):
                gather_copy(c, b, j).start()

        def drain(c, b):
            for j in range(n_sub):
                gather_copy(c, b, j).wait()

        def step(c, b, static):
            if static:
                if c >= 2:
                    store_copy(c - 2, (b + 2) % 4).wait()
                if c + 2 < sg:
                    fire(c + 2, (b + 2) % 4)
                if c + 3 < sg:
                    build_group(c + 3)
            else:
                store_copy(c - 2, (b + 2) % 4).wait()

                @pl.when(c + 2 < sg)
                def _():
                    fire(c + 2, (b + 2) % 4)

                @pl.when(c + 3 < sg)
                def _():
                    build_group(c + 3)

            drain(c, b)
            store_copy(c, b).start()

        build_group(0)
        build_group(1)
        build_group(2)
        fire(0, 0)
        fire(1, 1)
        step(0, 0, True)
        step(1, 1, True)

        def body(kk, carry):
            for u in range(4):
                step(2 + 4 * kk + u, (2 + u) % 4, False)
            return carry

        lax.fori_loop(0, (sg - 2) // 4, body, 0)
        store_copy(sg - 2, (sg - 2) % 4).wait()
        store_copy(sg - 1, (sg - 1) % 4).wait()

    return k


@functools.lru_cache(maxsize=None)
def _make_transpose(batch: int, seq: int, d: int):
    g = 128 // d
    sg = seq // g
    tb = batch // 128
    te = d // 8

    def body(in_ref, out_ref):
        for gi in range(sg):
            m = in_ref[0, pl.ds(gi * 128, 128), :]
            mt = jnp.transpose(m, (1, 0))
            out_ref[pl.ds(g * gi, g), :, 0, :, :] = mt.reshape(g, te, 8, 128)

    return pl.pallas_call(
        body,
        grid=(tb,),
        in_specs=[pl.BlockSpec((1, sg * 128, 128), lambda i: (i, 0, 0))],
        out_specs=pl.BlockSpec((seq, te, 1, 8, 128), lambda i: (0, 0, i, 0, 0)),
        out_shape=jax.ShapeDtypeStruct((seq, te, tb, 8, 128), jnp.float32),
    )


def kernel(x, embed_weight):
    b, s = x.shape
    n_tags, d = embed_weight.shape
    g = 128 // d
    sg = s // g
    tb = b // 128
    flat = x.reshape(b * s).astype(jnp.int32)
    tab = embed_weight.astype(jnp.float32)
    p1 = _make_gather(b * s, n_tags, d, s)(flat, tab)
    p3 = p1.reshape(tb, sg * 128, 128)
    o5 = _make_transpose(b, s, d)(p3)
    return o5.transpose(2, 4, 0, 1, 3).reshape(b, s, d)
